# baseline (speedup 1.0000x reference)
import functools

import jax
import jax.numpy as jnp
from jax import lax
from jax.experimental import pallas as pl
from jax.experimental.pallas import tpu as pltpu
from jax.experimental.pallas import tpu_sc as plsc

N_ROWS = 128
ROW = 32768
L = 16
NVEC = ROW // L
NUM_CORES = 2
NUM_SUBCORES = 16
WORKERS = NUM_CORES * NUM_SUBCORES
ROWS_PER_W = N_ROWS // WORKERS

S = 4
CHUNK = NVEC // S
STREAM_SHIFT = 13

B1, B2, B3, B4 = 256, 128, 512, 256


def _clear(hist, n, unroll=8):
    zeros = jnp.zeros((L,), jnp.int32)

    def body(i, c):
        hist[pl.ds(i * L, L)] = zeros
        return c

    lax.fori_loop(0, n // L, body, 0, unroll=unroll)


def _offsets(hist, nb):
    def body(v, carry):
        t = [hist[pl.ds(s * nb + v * L, L)] for s in range(S)]
        tot = t[0]
        for s in range(1, S):
            tot = tot + t[s]
        inc = plsc.cumsum(tot)
        run = inc - tot + carry
        hist[pl.ds(0 * nb + v * L, L)] = run
        for s in range(1, S):
            run = run + t[s - 1]
            hist[pl.ds(s * nb + v * L, L)] = run
        return carry + jnp.sum(tot)

    lax.fori_loop(0, nb // L, body, jnp.int32(0), unroll=2)


def _sc_body(in_hbm, out_hbm, buf_a, buf_b, buf_c, h1, h2, h3, h4):
    cid = lax.axis_index("c")
    sid = lax.axis_index("s")
    wid = sid * NUM_CORES + cid
    lane = lax.iota(jnp.int32, L)
    ones = jnp.ones((L,), jnp.int32)

    def do_row(j, c0):
        r = wid * ROWS_PER_W + j
        pltpu.sync_copy(in_hbm.at[r], buf_a)

        _clear(h1, S * B1)
        _clear(h2, S * B2)
        _clear(h3, S * B3)
        _clear(h4, S * B4)

        def hall(i, c):
            for s in range(S):
                vi = s * CHUNK + i
                u = plsc.bitcast(buf_a[pl.ds(vi * L, L)], jnp.int32)
                m = lax.shift_right_arithmetic(u, 31)
                k = u ^ ((m ^ jnp.int32(-1)) & jnp.int32(0x7FFFFFFF))
                buf_a[pl.ds(vi * L, L)] = plsc.bitcast(k, jnp.float32)
                plsc.addupdate_scatter(
                    h1, [(k & jnp.int32(0xFF)) + s * B1], ones)
            return c

        lax.fori_loop(0, CHUNK, hall, 0, unroll=2)
        _offsets(h1, B1)

        def s1(i, c):
            for s in range(S):
                vi = s * CHUNK + i
                k = plsc.bitcast(buf_a[pl.ds(vi * L, L)], jnp.int32)
                d = k & jnp.int32(0xFF)
                cnt, _ = plsc.scan_count(d)
                pos = plsc.load_gather(h1, [d + s * B1]) + cnt - 1
                plsc.addupdate_scatter(h1, [d + s * B1], ones)
                plsc.store_scatter(buf_b, [pos], k)
                plsc.store_scatter(buf_c, [pos], vi * L + lane)
                d2 = lax.shift_right_logical(k, 8) & jnp.int32(0x7F)
                dst = lax.shift_right_logical(pos, STREAM_SHIFT) * B2 + d2
                plsc.addupdate_scatter(h2, [dst], ones)
            return c

        lax.fori_loop(0, CHUNK, s1, 0)
        _offsets(h2, B2)

        def s2(i, c):
            for s in range(S):
                vi = s * CHUNK + i
                k = buf_b[pl.ds(vi * L, L)]
                idx = buf_c[pl.ds(vi * L, L)]
                d = lax.shift_right_logical(k, 8) & jnp.int32(0x7F)
                cnt, _ = plsc.scan_count(d)
                pos = plsc.load_gather(h2, [d + s * B2]) + cnt - 1
                plsc.addupdate_scatter(h2, [d + s * B2], ones)
                p = (k & jnp.int32(-32768)) | idx
                plsc.store_scatter(buf_a, [pos],
                                   plsc.bitcast(p, jnp.float32))
                d3 = lax.shift_right_logical(p, 15) & jnp.int32(0x1FF)
                dst = lax.shift_right_logical(pos, STREAM_SHIFT) * B3 + d3
                plsc.addupdate_scatter(h3, [dst], ones)
            return c

        lax.fori_loop(0, CHUNK, s2, 0)
        _offsets(h3, B3)

        def s3(i, c):
            for s in range(S):
                vi = s * CHUNK + i
                p = plsc.bitcast(buf_a[pl.ds(vi * L, L)], jnp.int32)
                d = lax.shift_right_logical(p, 15) & jnp.int32(0x1FF)
                cnt, _ = plsc.scan_count(d)
                pos = plsc.load_gather(h3, [d + s * B3]) + cnt - 1
                plsc.addupdate_scatter(h3, [d + s * B3], ones)
                plsc.store_scatter(buf_b, [pos], p)
                d4 = lax.shift_right_logical(p, 24) & jnp.int32(0xFF)
                dst = lax.shift_right_logical(pos, STREAM_SHIFT) * B4 + d4
                plsc.addupdate_scatter(h4, [dst], ones)
            return c

        lax.fori_loop(0, CHUNK, s3, 0)
        _offsets(h4, B4)

        def s4(i, c):
            for s in range(S):
                vi = s * CHUNK + i
                p = buf_b[pl.ds(vi * L, L)]
                d = lax.shift_right_logical(p, 24) & jnp.int32(0xFF)
                cnt, _ = plsc.scan_count(d)
                pos = plsc.load_gather(h4, [d + s * B4]) + cnt - 1
                plsc.addupdate_scatter(h4, [d + s * B4], ones)
                plsc.store_scatter(buf_c, [pos], p & jnp.int32(0x7FFF))
            return c

        lax.fori_loop(0, CHUNK, s4, 0)

        pltpu.sync_copy(buf_c, out_hbm.at[r])
        return c0

    lax.fori_loop(0, ROWS_PER_W, do_row, 0)


_argsort_desc = functools.partial(
    pl.kernel,
    out_type=jax.ShapeDtypeStruct((N_ROWS, ROW), jnp.int32),
    mesh=plsc.VectorSubcoreMesh(core_axis_name="c", subcore_axis_name="s"),
    scratch_types=[
        pltpu.VMEM((ROW,), jnp.float32),
        pltpu.VMEM((ROW,), jnp.int32),
        pltpu.VMEM((ROW,), jnp.int32),
        pltpu.VMEM((S * B1,), jnp.int32),
        pltpu.VMEM((S * B2,), jnp.int32),
        pltpu.VMEM((S * B3,), jnp.int32),
        pltpu.VMEM((S * B4,), jnp.int32),
    ],
    compiler_params=pltpu.CompilerParams(needs_layout_passes=False),
)(_sc_body)


@jax.jit
def kernel(inputs):
    return _argsort_desc(inputs)


# speedup vs baseline: 1.0210x; 1.0210x over previous
import functools

import jax
import jax.numpy as jnp
from jax import lax
from jax.experimental import pallas as pl
from jax.experimental.pallas import tpu as pltpu
from jax.experimental.pallas import tpu_sc as plsc

N_ROWS = 128
ROW = 32768
L = 16
NVEC = ROW // L
NUM_CORES = 2
NUM_SUBCORES = 16
WORKERS = NUM_CORES * NUM_SUBCORES
ROWS_PER_W = N_ROWS // WORKERS

S = 4
CHUNK = NVEC // S
STREAM_SHIFT = 13

B1, B2, B3, B4 = 256, 128, 512, 256


def _clear(ref, n, unroll=8):
    zeros = jnp.zeros((L,), jnp.int32)

    def body(i, c):
        ref[pl.ds(i * L, L)] = zeros
        return c

    lax.fori_loop(0, n // L, body, 0, unroll=unroll)


def _offsets_split(src_load, phs, nb):
    def body(v, carry):
        t = [src_load(s, v) for s in range(S)]
        tot = t[0]
        for s in range(1, S):
            tot = tot + t[s]
        inc = plsc.cumsum(tot)
        run = inc - tot + carry
        phs[0][pl.ds(v * L, L)] = run
        for s in range(1, S):
            run = run + t[s - 1]
            phs[s][pl.ds(v * L, L)] = run
        return carry + jnp.sum(tot)

    lax.fori_loop(0, nb // L, body, jnp.int32(0), unroll=2)


def _sc_body(in_hbm, out_hbm, buf_a, buf_b, buf_c,
             h1_0, h1_1, h1_2, h1_3, ch2, h2_0, h2_1, h2_2, h2_3,
             ch3, h3_0, h3_1, h3_2, h3_3, ch4, h4_0, h4_1, h4_2, h4_3):
    ph1 = [h1_0, h1_1, h1_2, h1_3]
    ph2 = [h2_0, h2_1, h2_2, h2_3]
    ph3 = [h3_0, h3_1, h3_2, h3_3]
    ph4 = [h4_0, h4_1, h4_2, h4_3]

    cid = lax.axis_index("c")
    sid = lax.axis_index("s")
    wid = sid * NUM_CORES + cid
    lane = lax.iota(jnp.int32, L)
    ones = jnp.ones((L,), jnp.int32)

    def do_row(j, c0):
        r = wid * ROWS_PER_W + j
        pltpu.sync_copy(in_hbm.at[r], buf_a)

        for s in range(S):
            _clear(ph1[s], B1)
        _clear(ch2, S * B2)
        _clear(ch3, S * B3)
        _clear(ch4, S * B4)

        def hall(i, c):
            for s in range(S):
                vi = s * CHUNK + i
                u = plsc.bitcast(buf_a[pl.ds(vi * L, L)], jnp.int32)
                m = lax.shift_right_arithmetic(u, 31)
                k = u ^ ((m ^ jnp.int32(-1)) & jnp.int32(0x7FFFFFFF))
                buf_a[pl.ds(vi * L, L)] = plsc.bitcast(k, jnp.float32)
                plsc.addupdate_scatter(ph1[s], [k & jnp.int32(0xFF)], ones)
            return c

        lax.fori_loop(0, CHUNK, hall, 0, unroll=2)
        _offsets_split(lambda s, v: ph1[s][pl.ds(v * L, L)], ph1, B1)

        def s1(i, c):
            for s in range(S):
                vi = s * CHUNK + i
                k = plsc.bitcast(buf_a[pl.ds(vi * L, L)], jnp.int32)
                d = k & jnp.int32(0xFF)
                cnt, _ = plsc.scan_count(d)
                pos = plsc.load_gather(ph1[s], [d]) + cnt - 1
                plsc.addupdate_scatter(ph1[s], [d], ones)
                plsc.store_scatter(buf_b, [pos], k)
                plsc.store_scatter(buf_c, [pos], vi * L + lane)
                d2 = lax.shift_right_logical(k, 8) & jnp.int32(0x7F)
                dst = lax.shift_right_logical(pos, STREAM_SHIFT) * B2 + d2
                plsc.addupdate_scatter(ch2, [dst], ones)
            return c

        lax.fori_loop(0, CHUNK, s1, 0)
        _offsets_split(lambda s, v: ch2[pl.ds(s * B2 + v * L, L)], ph2, B2)

        def s2(i, c):
            for s in range(S):
                vi = s * CHUNK + i
                k = buf_b[pl.ds(vi * L, L)]
                idx = buf_c[pl.ds(vi * L, L)]
                d = lax.shift_right_logical(k, 8) & jnp.int32(0x7F)
                cnt, _ = plsc.scan_count(d)
                pos = plsc.load_gather(ph2[s], [d]) + cnt - 1
                plsc.addupdate_scatter(ph2[s], [d], ones)
                p = (k & jnp.int32(-32768)) | idx
                plsc.store_scatter(buf_a, [pos],
                                   plsc.bitcast(p, jnp.float32))
                d3 = lax.shift_right_logical(p, 15) & jnp.int32(0x1FF)
                dst = lax.shift_right_logical(pos, STREAM_SHIFT) * B3 + d3
                plsc.addupdate_scatter(ch3, [dst], ones)
            return c

        lax.fori_loop(0, CHUNK, s2, 0)
        _offsets_split(lambda s, v: ch3[pl.ds(s * B3 + v * L, L)], ph3, B3)

        def s3(i, c):
            for s in range(S):
                vi = s * CHUNK + i
                p = plsc.bitcast(buf_a[pl.ds(vi * L, L)], jnp.int32)
                d = lax.shift_right_logical(p, 15) & jnp.int32(0x1FF)
                cnt, _ = plsc.scan_count(d)
                pos = plsc.load_gather(ph3[s], [d]) + cnt - 1
                plsc.addupdate_scatter(ph3[s], [d], ones)
                plsc.store_scatter(buf_b, [pos], p)
                d4 = lax.shift_right_logical(p, 24) & jnp.int32(0xFF)
                dst = lax.shift_right_logical(pos, STREAM_SHIFT) * B4 + d4
                plsc.addupdate_scatter(ch4, [dst], ones)
            return c

        lax.fori_loop(0, CHUNK, s3, 0)
        _offsets_split(lambda s, v: ch4[pl.ds(s * B4 + v * L, L)], ph4, B4)

        def s4(i, c):
            for s in range(S):
                vi = s * CHUNK + i
                p = buf_b[pl.ds(vi * L, L)]
                d = lax.shift_right_logical(p, 24) & jnp.int32(0xFF)
                cnt, _ = plsc.scan_count(d)
                pos = plsc.load_gather(ph4[s], [d]) + cnt - 1
                plsc.addupdate_scatter(ph4[s], [d], ones)
                plsc.store_scatter(buf_c, [pos], p & jnp.int32(0x7FFF))
            return c

        lax.fori_loop(0, CHUNK, s4, 0)

        pltpu.sync_copy(buf_c, out_hbm.at[r])
        return c0

    lax.fori_loop(0, ROWS_PER_W, do_row, 0)


_argsort_desc = functools.partial(
    pl.kernel,
    out_type=jax.ShapeDtypeStruct((N_ROWS, ROW), jnp.int32),
    mesh=plsc.VectorSubcoreMesh(core_axis_name="c", subcore_axis_name="s"),
    scratch_types=[
        pltpu.VMEM((ROW,), jnp.float32),
        pltpu.VMEM((ROW,), jnp.int32),
        pltpu.VMEM((ROW,), jnp.int32),
    ]
    + [pltpu.VMEM((B1,), jnp.int32) for _ in range(S)]
    + [pltpu.VMEM((S * B2,), jnp.int32)]
    + [pltpu.VMEM((B2,), jnp.int32) for _ in range(S)]
    + [pltpu.VMEM((S * B3,), jnp.int32)]
    + [pltpu.VMEM((B3,), jnp.int32) for _ in range(S)]
    + [pltpu.VMEM((S * B4,), jnp.int32)]
    + [pltpu.VMEM((B4,), jnp.int32) for _ in range(S)],
    compiler_params=pltpu.CompilerParams(needs_layout_passes=False),
)(_sc_body)


@jax.jit
def kernel(inputs):
    return _argsort_desc(inputs)


# speedup vs baseline: 1.8013x; 1.7642x over previous
import functools

import jax
import jax.numpy as jnp
from jax import lax
from jax.experimental import pallas as pl
from jax.experimental.pallas import tpu as pltpu
from jax.experimental.pallas import tpu_sc as plsc

N_ROWS = 128
ROW = 32768
L = 16
NVEC = ROW // L
NUM_CORES = 2
NUM_SUBCORES = 16
WORKERS = NUM_CORES * NUM_SUBCORES
ROWS_PER_W = N_ROWS // WORKERS


def _clear(hist, nvec, unroll=16):
    zeros = jnp.zeros((L,), jnp.int32)

    def body(i, c):
        hist[pl.ds(i * L, L)] = zeros
        return c

    lax.fori_loop(0, nvec, body, 0, unroll=unroll)


def _excl_prefix(hist, nvec, unroll=8):
    def body(i, carry):
        h = hist[pl.ds(i * L, L)]
        inc = plsc.cumsum(h)
        hist[pl.ds(i * L, L)] = inc - h + carry
        return carry + jnp.sum(h)

    lax.fori_loop(0, nvec, body, jnp.int32(0), unroll=unroll)


def _scatter_pass(src_load, digit_fn, payload_fn, store_fn, hist, ones,
                  unroll=8):
    def stage(i):
        x = src_load(i)
        d = digit_fn(x)
        cnt, _ = plsc.scan_count(d)
        return d, cnt, payload_fn(x, i)

    def commit(d, cnt, p):
        pos = plsc.load_gather(hist, [d]) + cnt - 1
        store_fn(pos, p)
        plsc.addupdate_scatter(hist, [d], ones)

    def body(i, carry):
        nxt = stage(i + 1)
        commit(*carry)
        return nxt

    last = lax.fori_loop(0, NVEC - 1, body, stage(0), unroll=unroll)
    commit(*last)


def _sc_body(in_hbm, out_hbm, buf_a, buf_b, hist1, hist2, hist3):
    cid = lax.axis_index("c")
    sid = lax.axis_index("s")
    wid = sid * NUM_CORES + cid
    lane = lax.iota(jnp.int32, L)
    ones = jnp.ones((L,), jnp.int32)

    def do_row(j, c0):
        r = wid * ROWS_PER_W + j
        pltpu.sync_copy(in_hbm.at[r], buf_a)

        _clear(hist1, NVEC)
        _clear(hist2, 512 // L)
        _clear(hist3, 256 // L)

        def hall(i, c):
            v = buf_a[pl.ds(i * L, L)]
            u = plsc.bitcast(v, jnp.int32)
            m = lax.shift_right_arithmetic(u, 31)
            k = u ^ ((m ^ jnp.int32(-1)) & jnp.int32(0x7FFFFFFF))
            buf_a[pl.ds(i * L, L)] = plsc.bitcast(k, jnp.float32)
            plsc.addupdate_scatter(hist1, [k & jnp.int32(0x7FFF)], ones)
            plsc.addupdate_scatter(
                hist2, [lax.shift_right_logical(k, 15) & jnp.int32(0x1FF)],
                ones)
            plsc.addupdate_scatter(
                hist3, [lax.shift_right_logical(k, 24) & jnp.int32(0xFF)],
                ones)
            return c

        lax.fori_loop(0, NVEC, hall, 0, unroll=8)

        _excl_prefix(hist1, NVEC)
        _excl_prefix(hist2, 512 // L)
        _excl_prefix(hist3, 256 // L)

        _scatter_pass(
            src_load=lambda i: plsc.bitcast(buf_a[pl.ds(i * L, L)], jnp.int32),
            digit_fn=lambda k: k & jnp.int32(0x7FFF),
            payload_fn=lambda k, i: (k & jnp.int32(-32768)) | (i * L + lane),
            store_fn=lambda pos, p: plsc.store_scatter(buf_b, [pos], p),
            hist=hist1, ones=ones)

        _scatter_pass(
            src_load=lambda i: buf_b[pl.ds(i * L, L)],
            digit_fn=lambda p: lax.shift_right_logical(p, 15)
            & jnp.int32(0x1FF),
            payload_fn=lambda p, i: p,
            store_fn=lambda pos, p: plsc.store_scatter(
                buf_a, [pos], plsc.bitcast(p, jnp.float32)),
            hist=hist2, ones=ones)

        _scatter_pass(
            src_load=lambda i: plsc.bitcast(buf_a[pl.ds(i * L, L)], jnp.int32),
            digit_fn=lambda p: lax.shift_right_logical(p, 24)
            & jnp.int32(0xFF),
            payload_fn=lambda p, i: p & jnp.int32(0x7FFF),
            store_fn=lambda pos, p: plsc.store_scatter(buf_b, [pos], p),
            hist=hist3, ones=ones)

        pltpu.sync_copy(buf_b, out_hbm.at[r])
        return c0

    lax.fori_loop(0, ROWS_PER_W, do_row, 0)


_argsort_desc = functools.partial(
    pl.kernel,
    out_type=jax.ShapeDtypeStruct((N_ROWS, ROW), jnp.int32),
    mesh=plsc.VectorSubcoreMesh(core_axis_name="c", subcore_axis_name="s"),
    scratch_types=[
        pltpu.VMEM((ROW,), jnp.float32),
        pltpu.VMEM((ROW,), jnp.int32),
        pltpu.VMEM((ROW,), jnp.int32),
        pltpu.VMEM((512,), jnp.int32),
        pltpu.VMEM((256,), jnp.int32),
    ],
    compiler_params=pltpu.CompilerParams(needs_layout_passes=False),
)(_sc_body)


@jax.jit
def kernel(inputs):
    return _argsort_desc(inputs)


# speedup vs baseline: 2.2201x; 1.2325x over previous
import functools

import jax
import jax.numpy as jnp
from jax import lax
from jax.experimental import pallas as pl
from jax.experimental.pallas import tpu as pltpu
from jax.experimental.pallas import tpu_sc as plsc

N_ROWS = 128
ROW = 32768
L = 16
NVEC = ROW // L
NUM_CORES = 2
NUM_SUBCORES = 16
WORKERS = NUM_CORES * NUM_SUBCORES
ROWS_PER_W = N_ROWS // WORKERS


def _clear(hist, nvec, unroll=16):
    zeros = jnp.zeros((L,), jnp.int32)

    @plsc.parallel_loop(0, nvec, unroll=unroll)
    def _(i):
        hist[pl.ds(i * L, L)] = zeros


def _excl_prefix(hist, nvec, unroll=8):
    @plsc.parallel_loop(0, nvec, unroll=unroll, carry=jnp.int32(0))
    def _(i, carry):
        h = hist[pl.ds(i * L, L)]
        inc = plsc.cumsum(h)
        hist[pl.ds(i * L, L)] = inc - h + carry
        return carry + jnp.sum(h)


def _scatter_pass(src_load, digit_fn, payload_fn, store_fn, hist, ones,
                  unroll=4):
    def stage(i):
        x = src_load(i)
        d = digit_fn(x)
        cnt, _ = plsc.scan_count(d)
        return d, cnt, payload_fn(x, i)

    def commit(d, cnt, p):
        pos = plsc.load_gather(hist, [d]) + cnt - 1
        store_fn(pos, p)
        plsc.addupdate_scatter(hist, [d], ones)

    def body(i, carry):
        nxt = stage(i + 1)
        commit(*carry)
        return nxt

    last = lax.fori_loop(0, NVEC - 1, body, stage(0), unroll=unroll)
    commit(*last)


def _sc_body(in_hbm, out_hbm, buf_a, buf_b, hist1, hist2, hist3):
    cid = lax.axis_index("c")
    sid = lax.axis_index("s")
    wid = sid * NUM_CORES + cid
    lane = lax.iota(jnp.int32, L)
    ones = jnp.ones((L,), jnp.int32)

    def do_row(j, c0):
        r = wid * ROWS_PER_W + j
        pltpu.sync_copy(in_hbm.at[r], buf_a)

        _clear(hist1, NVEC)
        _clear(hist2, 512 // L)
        _clear(hist3, 256 // L)

        def hall(i, c):
            v = buf_a[pl.ds(i * L, L)]
            u = plsc.bitcast(v, jnp.int32)
            m = lax.shift_right_arithmetic(u, 31)
            k = u ^ ((m ^ jnp.int32(-1)) & jnp.int32(0x7FFFFFFF))
            buf_a[pl.ds(i * L, L)] = plsc.bitcast(k, jnp.float32)
            plsc.addupdate_scatter(hist1, [k & jnp.int32(0x7FFF)], ones)
            plsc.addupdate_scatter(
                hist2, [lax.shift_right_logical(k, 15) & jnp.int32(0x1FF)],
                ones)
            plsc.addupdate_scatter(
                hist3, [lax.shift_right_logical(k, 24) & jnp.int32(0xFF)],
                ones)
            return c

        plsc.parallel_loop(0, NVEC, unroll=4)(
            lambda i: hall(i, 0) and None)

        _excl_prefix(hist1, NVEC)
        _excl_prefix(hist2, 512 // L)
        _excl_prefix(hist3, 256 // L)

        _scatter_pass(
            src_load=lambda i: plsc.bitcast(buf_a[pl.ds(i * L, L)], jnp.int32),
            digit_fn=lambda k: k & jnp.int32(0x7FFF),
            payload_fn=lambda k, i: (k & jnp.int32(-32768)) | (i * L + lane),
            store_fn=lambda pos, p: plsc.store_scatter(buf_b, [pos], p),
            hist=hist1, ones=ones)

        _scatter_pass(
            src_load=lambda i: buf_b[pl.ds(i * L, L)],
            digit_fn=lambda p: lax.shift_right_logical(p, 15)
            & jnp.int32(0x1FF),
            payload_fn=lambda p, i: p,
            store_fn=lambda pos, p: plsc.store_scatter(
                buf_a, [pos], plsc.bitcast(p, jnp.float32)),
            hist=hist2, ones=ones)

        _scatter_pass(
            src_load=lambda i: plsc.bitcast(buf_a[pl.ds(i * L, L)], jnp.int32),
            digit_fn=lambda p: lax.shift_right_logical(p, 24)
            & jnp.int32(0xFF),
            payload_fn=lambda p, i: p & jnp.int32(0x7FFF),
            store_fn=lambda pos, p: plsc.store_scatter(buf_b, [pos], p),
            hist=hist3, ones=ones)

        pltpu.sync_copy(buf_b, out_hbm.at[r])
        return c0

    lax.fori_loop(0, ROWS_PER_W, do_row, 0)


_argsort_desc = functools.partial(
    pl.kernel,
    out_type=jax.ShapeDtypeStruct((N_ROWS, ROW), jnp.int32),
    mesh=plsc.VectorSubcoreMesh(core_axis_name="c", subcore_axis_name="s"),
    scratch_types=[
        pltpu.VMEM((ROW,), jnp.float32),
        pltpu.VMEM((ROW,), jnp.int32),
        pltpu.VMEM((ROW,), jnp.int32),
        pltpu.VMEM((512,), jnp.int32),
        pltpu.VMEM((256,), jnp.int32),
    ],
    compiler_params=pltpu.CompilerParams(needs_layout_passes=False),
)(_sc_body)


@jax.jit
def kernel(inputs):
    return _argsort_desc(inputs)
